# Initial kernel scaffold; baseline (speedup 1.0000x reference)
import functools

import jax
import jax.numpy as jnp
from jax import lax
from jax.experimental import pallas as pl
from jax.experimental.pallas import tpu as pltpu
from jax.experimental.pallas import tpu_sc as plsc

VOCAB = 100000
EMBED_DIM = 300
MAX_NORM = 1.0
CTX = 200

NC = 2
NS = 16
NW = NC * NS
B_PAD = 256
B_PER_W = B_PAD // NW

BLK = 2500


def _sc_gather_body(table_hbm, idx_hbm, out_hbm, idx_v, rows_v, sem):
    wid = lax.axis_index("s") * NC + lax.axis_index("c")
    base = wid * B_PER_W
    pltpu.sync_copy(idx_hbm.at[pl.ds(base, B_PER_W)], idx_v)
    pltpu.async_copy(table_hbm.at[idx_v], rows_v, sem).wait()
    pltpu.sync_copy(rows_v, out_hbm.at[pl.ds(base, B_PER_W)])


_sc_gather = functools.partial(
    pl.kernel,
    mesh=plsc.VectorSubcoreMesh(core_axis_name="c", subcore_axis_name="s"),
    out_type=jax.ShapeDtypeStruct((B_PAD, EMBED_DIM), jnp.float32),
    scratch_types=[
        pltpu.VMEM((B_PER_W,), jnp.int32),
        pltpu.VMEM((B_PER_W, EMBED_DIM), jnp.float32),
        pltpu.SemaphoreType.DMA,
    ],
)(_sc_gather_body)


def _tc_body(g_ref, w_ref, b_ref, out_ref, avg_ref):
    @pl.when(pl.program_id(0) == 0)
    def _():
        g = g_ref[...]
        ss = jnp.sum(g * g, axis=1, keepdims=True)
        norm = jnp.sqrt(ss)
        scale = jnp.minimum(1.0, MAX_NORM / jnp.maximum(norm, 1e-7))
        avg_ref[...] = jnp.sum(g * scale, axis=0, keepdims=True) * (1.0 / CTX)

    out_ref[...] = (
        lax.dot_general(
            avg_ref[...], w_ref[...],
            (((1,), (1,)), ((), ())),
            preferred_element_type=jnp.float32,
        )
        + b_ref[...]
    )


def kernel(inputs, emb_table, W, b):
    idx = jnp.zeros((B_PAD,), jnp.int32).at[:CTX].set(inputs.astype(jnp.int32))
    gathered = _sc_gather(emb_table, idx)

    logits = pl.pallas_call(
        _tc_body,
        grid=(VOCAB // BLK,),
        in_specs=[
            pl.BlockSpec((CTX, EMBED_DIM), lambda i: (0, 0)),
            pl.BlockSpec((BLK, EMBED_DIM), lambda i: (i, 0)),
            pl.BlockSpec((1, BLK), lambda i: (0, i)),
        ],
        out_specs=pl.BlockSpec((1, BLK), lambda i: (0, i)),
        out_shape=jax.ShapeDtypeStruct((1, VOCAB), jnp.float32),
        scratch_shapes=[pltpu.VMEM((1, EMBED_DIM), jnp.float32)],
    )(gathered[:CTX], W, b.reshape(1, VOCAB))
    return logits


# speedup vs baseline: 1.6258x; 1.6258x over previous
import functools

import jax
import jax.numpy as jnp
from jax import lax
from jax.experimental import pallas as pl
from jax.experimental.pallas import tpu as pltpu
from jax.experimental.pallas import tpu_sc as plsc

VOCAB = 100000
EMBED_DIM = 300
MAX_NORM = 1.0
CTX = 200

NC = 2
NS = 16
NW = NC * NS
B_PAD = 256
B_PER_W = B_PAD // NW

BLK = 2000


def _sc_gather_body(table_hbm, idx_hbm, out_hbm, idx_v, rows_v, sem):
    wid = lax.axis_index("s") * NC + lax.axis_index("c")
    base = wid * B_PER_W
    pltpu.sync_copy(idx_hbm.at[pl.ds(base, 16)], idx_v)
    rows = idx_v[...]
    for j in range(B_PER_W):
        pltpu.make_async_copy(
            table_hbm.at[pl.ds(rows[j], 1)], rows_v.at[pl.ds(j, 1)], sem
        ).start()
    for _ in range(B_PER_W):
        pltpu.make_async_copy(
            table_hbm.at[pl.ds(0, 1)], rows_v.at[pl.ds(0, 1)], sem
        ).wait()
    pltpu.sync_copy(rows_v, out_hbm.at[pl.ds(base, B_PER_W)])


@functools.lru_cache(maxsize=1)
def _make_sc_gather():
    return functools.partial(
        pl.kernel,
        mesh=plsc.VectorSubcoreMesh(core_axis_name="c", subcore_axis_name="s"),
        out_type=jax.ShapeDtypeStruct((B_PAD, EMBED_DIM), jnp.float32),
        scratch_types=[
            pltpu.VMEM((16,), jnp.int32),
            pltpu.VMEM((B_PER_W, EMBED_DIM), jnp.float32),
            pltpu.SemaphoreType.DMA,
        ],
    )(_sc_gather_body)


def _tc_body(g_ref, w_ref, b_ref, out_ref, avg_ref):
    @pl.when(pl.program_id(0) == 0)
    def _():
        g = g_ref[...]
        ss = jnp.sum(g * g, axis=1, keepdims=True)
        norm = jnp.sqrt(ss)
        scale = jnp.minimum(1.0, MAX_NORM / jnp.maximum(norm, 1e-7))
        avg_ref[...] = jnp.sum(g * scale, axis=0, keepdims=True) * (1.0 / CTX)

    res = lax.dot_general(
        avg_ref[...], w_ref[...],
        (((1,), (1,)), ((), ())),
        preferred_element_type=jnp.float32,
    )
    out_ref[...] = res.reshape(1, 1, BLK) + b_ref[...]


def kernel(inputs, emb_table, W, b):
    idx = jnp.zeros((B_PAD + 8,), jnp.int32).at[:CTX].set(inputs.astype(jnp.int32))
    gathered = _make_sc_gather()(emb_table, idx)

    nblk = VOCAB // BLK
    logits = pl.pallas_call(
        _tc_body,
        grid=(nblk,),
        in_specs=[
            pl.BlockSpec((CTX, EMBED_DIM), lambda i: (0, 0)),
            pl.BlockSpec((BLK, EMBED_DIM), lambda i: (i, 0)),
            pl.BlockSpec((1, 1, BLK), lambda i: (i, 0, 0)),
        ],
        out_specs=pl.BlockSpec((1, 1, BLK), lambda i: (i, 0, 0)),
        out_shape=jax.ShapeDtypeStruct((nblk, 1, BLK), jnp.float32),
        scratch_shapes=[pltpu.VMEM((1, EMBED_DIM), jnp.float32)],
    )(gathered[:CTX], W, b.reshape(nblk, 1, BLK))
    return logits.reshape(1, VOCAB)


# speedup vs baseline: 1.6991x; 1.0451x over previous
import functools

import jax
import jax.numpy as jnp
from jax import lax
from jax.experimental import pallas as pl
from jax.experimental.pallas import tpu as pltpu
from jax.experimental.pallas import tpu_sc as plsc

VOCAB = 100000
EMBED_DIM = 300
MAX_NORM = 1.0
CTX = 200

NC = 2
NS = 16
NW = NC * NS
B_PAD = 256
B_PER_W = B_PAD // NW

BLK = 4000


def _sc_gather_body(table_hbm, idx_hbm, out_hbm, idx_v, rows_v, sem):
    wid = lax.axis_index("s") * NC + lax.axis_index("c")
    base = wid * B_PER_W
    pltpu.sync_copy(idx_hbm.at[pl.ds(base, 16)], idx_v)
    rows = idx_v[...]
    for j in range(B_PER_W):
        pltpu.make_async_copy(
            table_hbm.at[pl.ds(rows[j], 1)], rows_v.at[pl.ds(j, 1)], sem
        ).start()
    for _ in range(B_PER_W):
        pltpu.make_async_copy(
            table_hbm.at[pl.ds(0, 1)], rows_v.at[pl.ds(0, 1)], sem
        ).wait()
    pltpu.sync_copy(rows_v, out_hbm.at[pl.ds(base, B_PER_W)])


@functools.lru_cache(maxsize=1)
def _make_sc_gather():
    return functools.partial(
        pl.kernel,
        mesh=plsc.VectorSubcoreMesh(core_axis_name="c", subcore_axis_name="s"),
        out_type=jax.ShapeDtypeStruct((B_PAD, EMBED_DIM), jnp.float32),
        scratch_types=[
            pltpu.VMEM((16,), jnp.int32),
            pltpu.VMEM((B_PER_W, EMBED_DIM), jnp.float32),
            pltpu.SemaphoreType.DMA,
        ],
    )(_sc_gather_body)


def _tc_body(g_ref, w_ref, b_ref, out_ref, avg_ref):
    @pl.when(pl.program_id(0) == 0)
    def _():
        g = g_ref[...]
        ss = jnp.sum(g * g, axis=1, keepdims=True)
        norm = jnp.sqrt(ss)
        scale = jnp.minimum(1.0, MAX_NORM / jnp.maximum(norm, 1e-7))
        avg_ref[...] = jnp.sum(g * scale, axis=0, keepdims=True) * (1.0 / CTX)

    res = lax.dot_general(
        avg_ref[...], w_ref[...],
        (((1,), (1,)), ((), ())),
        preferred_element_type=jnp.float32,
    )
    out_ref[...] = res.reshape(1, 1, BLK) + b_ref[...]


def kernel(inputs, emb_table, W, b):
    idx = jnp.zeros((B_PAD + 8,), jnp.int32).at[:CTX].set(inputs.astype(jnp.int32))
    gathered = _make_sc_gather()(emb_table, idx)

    nblk = VOCAB // BLK
    logits = pl.pallas_call(
        _tc_body,
        grid=(nblk,),
        in_specs=[
            pl.BlockSpec((CTX, EMBED_DIM), lambda i: (0, 0)),
            pl.BlockSpec((BLK, EMBED_DIM), lambda i: (i, 0)),
            pl.BlockSpec((1, 1, BLK), lambda i: (i, 0, 0)),
        ],
        out_specs=pl.BlockSpec((1, 1, BLK), lambda i: (i, 0, 0)),
        out_shape=jax.ShapeDtypeStruct((nblk, 1, BLK), jnp.float32),
        scratch_shapes=[pltpu.VMEM((1, EMBED_DIM), jnp.float32)],
    )(gathered[:CTX], W, b.reshape(nblk, 1, BLK))
    return logits.reshape(1, VOCAB)
